# baseline (speedup 1.0000x reference)
import functools

import jax
import jax.numpy as jnp
from jax import lax
from jax.experimental import pallas as pl
from jax.experimental.pallas import tpu as pltpu
from jax.experimental.pallas import tpu_sc as plsc

_B = 16384
_C = 1000
_NC = 2
_NS = 16
_NW = _NC * _NS
_L = 16
_BPW = _B // _NW
_G = 128
_NG = _BPW // _G
_CHUNKS = _BPW // _L

_mesh = plsc.VectorSubcoreMesh(core_axis_name="c", subcore_axis_name="s")


@functools.partial(
    pl.kernel,
    mesh=_mesh,
    compiler_params=pltpu.CompilerParams(needs_layout_passes=False),
    out_type=jax.ShapeDtypeStruct((_B,), jnp.float32),
    scratch_types=[
        pltpu.VMEM((_BPW,), jnp.int32),
        pltpu.VMEM((_BPW, _G), jnp.float32),
        pltpu.VMEM((_BPW,), jnp.float32),
        pltpu.SemaphoreType.DMA,
    ],
)
def _depth_sc(xt_hbm, labels_hbm, out_hbm, lab_v, seg_v, out_v, sem):
    wid = lax.axis_index("s") * _NC + lax.axis_index("c")
    base = wid * _BPW

    pltpu.sync_copy(labels_hbm.at[pl.ds(base, _BPW)], lab_v)

    copies = [
        pltpu.make_async_copy(
            xt_hbm.at[
                plsc.Indices(lab_v.at[pl.ds(c * _G, _G)]),
                pl.ds(base + c * _G, _G),
            ],
            seg_v.at[pl.ds(c * _G, _G), :],
            sem,
        )
        for c in range(_NG)
    ]
    for cp in copies:
        cp.start()
    for cp in copies:
        cp.wait()

    lane = lax.iota(jnp.int32, _L)

    def extract(j, _):
        pos = j * _L + lane
        v = plsc.load_gather(seg_v, [pos, pos & (_G - 1)])
        out_v[pl.ds(j * _L, _L)] = jnp.exp(jnp.abs(v * jnp.float32(0.1))) - 1.0
        return _

    lax.fori_loop(0, _CHUNKS, extract, None)

    pltpu.sync_copy(out_v, out_hbm.at[pl.ds(base, _BPW)])


def kernel(x, labels):
    depth = _depth_sc(x.T, labels.astype(jnp.int32))
    return depth[:, None]
